# baseline (speedup 1.0000x reference)
import functools
import jax
import jax.numpy as jnp
from jax import lax
from jax.experimental import pallas as pl
from jax.experimental.pallas import tpu as pltpu
from jax.experimental.pallas import tpu_sc as plsc

_L = 16


def _sc_add(x_hbm, emb_hbm, out_hbm, xbuf, ebuf):
    batch, seq_len, d_model = x_hbm.shape
    n_workers = 32
    s_per_w = seq_len // n_workers
    ch = xbuf.shape[0]
    n_ch = s_per_w // ch

    wid = lax.axis_index("s") * 2 + lax.axis_index("c")
    s_base = wid * s_per_w

    def chunk_body(c, _):
        s0 = s_base + c * ch
        pltpu.sync_copy(emb_hbm.at[pl.ds(s0, ch)], ebuf)

        def batch_body(b, _):
            pltpu.sync_copy(x_hbm.at[b, pl.ds(s0, ch)], xbuf)

            def row_body(r, _):
                def col_body(j, _):
                    sl = pl.ds(j * _L, _L)
                    xbuf[r, sl] = xbuf[r, sl] + ebuf[r, sl]
                    return 0

                return lax.fori_loop(0, d_model // _L, col_body, 0)

            lax.fori_loop(0, ch, row_body, 0)
            pltpu.sync_copy(xbuf, out_hbm.at[b, pl.ds(s0, ch)])
            return 0

        lax.fori_loop(0, batch, batch_body, 0)
        return 0

    lax.fori_loop(0, n_ch, chunk_body, 0)


def kernel(x, emb_weight):
    batch, seq_len, d_model = x.shape
    ch = 16
    run = functools.partial(
        pl.kernel,
        mesh=plsc.VectorSubcoreMesh(core_axis_name="c", subcore_axis_name="s"),
        out_type=jax.ShapeDtypeStruct((batch, seq_len, d_model), x.dtype),
        scratch_types=[
            pltpu.VMEM((ch, d_model), jnp.float32),
            pltpu.VMEM((ch, d_model), jnp.float32),
        ],
    )(_sc_add)
    return run(x, emb_weight)


# speedup vs baseline: 1.2127x; 1.2127x over previous
import functools
import jax
import jax.numpy as jnp
from jax import lax
from jax.experimental import pallas as pl
from jax.experimental.pallas import tpu as pltpu
from jax.experimental.pallas import tpu_sc as plsc

_L = 16
_CH = 16


def _sc_add(x_hbm, emb_hbm, out_hbm, xb, eb, sx0, sx1, se0, se1, ss0, ss1):
    batch, seq_len, d_model = x_hbm.shape
    n_workers = 32
    s_per_w = seq_len // n_workers
    n_ch = s_per_w // _CH
    n_it = n_ch * batch

    wid = lax.axis_index("s") * 2 + lax.axis_index("c")
    s_base = wid * s_per_w

    semx = [sx0, sx1]
    seme = [se0, se1]
    sems = [ss0, ss1]

    def s0_of(c):
        return s_base + c * _CH

    def x_slice(i):
        return (i % batch, pl.ds(s0_of(i // batch), _CH))

    ecp = [None, None]
    xcp = [None, None]
    scp = [None, None]
    ecp[0] = pltpu.async_copy(emb_hbm.at[pl.ds(s0_of(0), _CH)], eb.at[0], seme[0])
    b0, sl0 = x_slice(0)
    xcp[0] = pltpu.async_copy(x_hbm.at[b0, sl0], xb.at[0], semx[0])

    for c in range(n_ch):
        ec = c % 2
        if c + 1 < n_ch:
            ecp[(c + 1) % 2] = pltpu.async_copy(
                emb_hbm.at[pl.ds(s0_of(c + 1), _CH)], eb.at[(c + 1) % 2],
                seme[(c + 1) % 2])
        ecp[ec].wait()
        for b in range(batch):
            i = c * batch + b
            slot = i % 2
            nxt = (i + 1) % 2
            if i + 1 < n_it:
                if scp[nxt] is not None:
                    scp[nxt].wait()
                    scp[nxt] = None
                bn, sln = x_slice(i + 1)
                xcp[nxt] = pltpu.async_copy(x_hbm.at[bn, sln], xb.at[nxt],
                                            semx[nxt])
            xcp[slot].wait()

            def row_body(r, _):
                def col_body(j, _):
                    sl = pl.ds(j * _L, _L)
                    xb[slot, r, sl] = xb[slot, r, sl] + eb[ec, r, sl]
                    return 0

                return lax.fori_loop(0, d_model // _L, col_body, 0)

            lax.fori_loop(0, _CH, row_body, 0)
            if scp[slot] is not None:
                scp[slot].wait()
                scp[slot] = None
            bi, sli = x_slice(i)
            scp[slot] = pltpu.async_copy(xb.at[slot], out_hbm.at[bi, sli],
                                         sems[slot])
    for k in range(2):
        if scp[k] is not None:
            scp[k].wait()


def kernel(x, emb_weight):
    batch, seq_len, d_model = x.shape
    run = functools.partial(
        pl.kernel,
        mesh=plsc.VectorSubcoreMesh(core_axis_name="c", subcore_axis_name="s"),
        out_type=jax.ShapeDtypeStruct((batch, seq_len, d_model), x.dtype),
        scratch_types=[
            pltpu.VMEM((2, _CH, d_model), jnp.float32),
            pltpu.VMEM((2, _CH, d_model), jnp.float32),
            pltpu.SemaphoreType.DMA,
            pltpu.SemaphoreType.DMA,
            pltpu.SemaphoreType.DMA,
            pltpu.SemaphoreType.DMA,
            pltpu.SemaphoreType.DMA,
            pltpu.SemaphoreType.DMA,
        ],
    )(_sc_add)
    return run(x, emb_weight)
